# baseline (device time: 379016 ns/iter reference)
import jax
import jax.numpy as jnp
from jax import lax
from jax.experimental import pallas as pl
from jax.experimental.pallas import tpu as pltpu

N_DEV = 8
SUB = 4
NSLOT = 3


def kernel(x, w_mat):
    m, k = x.shape
    _, n = w_mat.shape
    m_per = m // N_DEV
    half = m_per // 2
    rows = half // SUB
    x = x.astype(jnp.bfloat16)
    w_mat = w_mat.astype(jnp.bfloat16)

    def body(x_ref, w_ref, out_ref, comm_r, comm_l,
             send_r, recv_r, send_l, recv_l, credit_r, credit_l):
        my = lax.axis_index("i")
        left = lax.rem(my + N_DEV - 1, N_DEV)
        right = lax.rem(my + 1, N_DEV)

        barrier_sem = pltpu.get_barrier_semaphore()
        for nbr in (left, right):
            pl.semaphore_signal(
                barrier_sem, inc=1,
                device_id=(nbr,), device_id_type=pl.DeviceIdType.MESH,
            )
        pl.semaphore_wait(barrier_sem, 2)

        def partial_top(c, j=None):
            start = c * m_per if j is None else c * m_per + j * rows
            xc = x_ref[pl.ds(start, half if j is None else rows), :]
            return jnp.dot(xc, w_ref[:, :], preferred_element_type=jnp.float32)

        def partial_bot(c, j=None):
            start = c * m_per + half if j is None else c * m_per + half + j * rows
            xc = x_ref[pl.ds(start, half if j is None else rows), :]
            return jnp.dot(xc, w_ref[:, :], preferred_element_type=jnp.float32)

        def sub_copy(comm, slot_s, slot_r, j, ssem, rsem, dev):
            return pltpu.make_async_remote_copy(
                src_ref=comm.at[slot_s, pl.ds(j * rows, rows), :],
                dst_ref=comm.at[slot_r, pl.ds(j * rows, rows), :],
                send_sem=ssem, recv_sem=rsem,
                device_id=(dev,), device_id_type=pl.DeviceIdType.MESH,
            )

        for s in range(N_DEV - 1):
            snd = (s + NSLOT - 1) % NSLOT
            rcv = s % NSLOT
            if s >= 2:
                pl.semaphore_wait(credit_r.at[rcv], 1)
                pl.semaphore_wait(credit_l.at[rcv], 1)
            sends = []
            for j in range(SUB):
                jr = pl.ds(j * rows, rows)
                if s == 0:
                    comm_r[snd, jr, :] = partial_top(left, j).astype(jnp.bfloat16)
                    comm_l[snd, jr, :] = partial_bot(right, j).astype(jnp.bfloat16)
                r = sub_copy(comm_r, snd, rcv, j, send_r.at[snd, j],
                             recv_r.at[rcv, j], right)
                l = sub_copy(comm_l, snd, rcv, j, send_l.at[snd, j],
                             recv_l.at[rcv, j], left)
                r.start()
                l.start()
                sends.append((r, l))
            p_top = partial_top(
                lax.rem(my + 2 * N_DEV - s - 2, N_DEV)).astype(jnp.bfloat16)
            p_bot = partial_bot(
                lax.rem(my + s + 2, N_DEV)).astype(jnp.bfloat16)
            for j in range(SUB):
                jr = pl.ds(j * rows, rows)
                sub_copy(comm_r, rcv, rcv, j, send_r.at[rcv, j],
                         recv_r.at[rcv, j], left).wait_recv()
                acc_top = (comm_r[rcv, jr, :].astype(jnp.float32)
                           + p_top[j * rows:(j + 1) * rows, :].astype(jnp.float32))
                sub_copy(comm_l, rcv, rcv, j, send_l.at[rcv, j],
                         recv_l.at[rcv, j], right).wait_recv()
                acc_bot = (comm_l[rcv, jr, :].astype(jnp.float32)
                           + p_bot[j * rows:(j + 1) * rows, :].astype(jnp.float32))
                if s < N_DEV - 2:
                    comm_r[rcv, jr, :] = acc_top.astype(jnp.bfloat16)
                    comm_l[rcv, jr, :] = acc_bot.astype(jnp.bfloat16)
                else:
                    out_ref[jr, :] = acc_top
                    out_ref[pl.ds(half + j * rows, rows), :] = acc_bot
            for r, l in sends:
                r.wait_send()
                l.wait_send()
            if s < N_DEV - 3:
                pl.semaphore_signal(
                    credit_r.at[snd], inc=1,
                    device_id=(left,), device_id_type=pl.DeviceIdType.MESH,
                )
                pl.semaphore_signal(
                    credit_l.at[snd], inc=1,
                    device_id=(right,), device_id_type=pl.DeviceIdType.MESH,
                )

    return pl.pallas_call(
        body,
        out_shape=jax.ShapeDtypeStruct((m_per, n), jnp.float32),
        in_specs=[
            pl.BlockSpec(memory_space=pltpu.VMEM),
            pl.BlockSpec(memory_space=pltpu.VMEM),
        ],
        out_specs=pl.BlockSpec(memory_space=pltpu.VMEM),
        scratch_shapes=[
            pltpu.VMEM((NSLOT, half, n), jnp.bfloat16),
            pltpu.VMEM((NSLOT, half, n), jnp.bfloat16),
            pltpu.SemaphoreType.DMA((NSLOT, SUB)),
            pltpu.SemaphoreType.DMA((NSLOT, SUB)),
            pltpu.SemaphoreType.DMA((NSLOT, SUB)),
            pltpu.SemaphoreType.DMA((NSLOT, SUB)),
            pltpu.SemaphoreType.REGULAR((NSLOT,)),
            pltpu.SemaphoreType.REGULAR((NSLOT,)),
        ],
        compiler_params=pltpu.CompilerParams(
            collective_id=0,
            vmem_limit_bytes=100 * 1024 * 1024,
        ),
    )(x, w_mat)


# device time: 367792 ns/iter; 1.0305x vs baseline; 1.0305x over previous
import jax
import jax.numpy as jnp
from jax import lax
from jax.experimental import pallas as pl
from jax.experimental.pallas import tpu as pltpu

N_DEV = 8
SUB = 8
NSLOT = 3


def kernel(x, w_mat):
    m, k = x.shape
    _, n = w_mat.shape
    m_per = m // N_DEV
    half = m_per // 2
    rows = half // SUB
    x = x.astype(jnp.bfloat16)
    w_mat = w_mat.astype(jnp.bfloat16)

    def body(x_ref, w_ref, out_ref, comm_r, comm_l,
             send_r, recv_r, send_l, recv_l, credit_r, credit_l):
        my = lax.axis_index("i")
        left = lax.rem(my + N_DEV - 1, N_DEV)
        right = lax.rem(my + 1, N_DEV)

        barrier_sem = pltpu.get_barrier_semaphore()
        for nbr in (left, right):
            pl.semaphore_signal(
                barrier_sem, inc=1,
                device_id=(nbr,), device_id_type=pl.DeviceIdType.MESH,
            )
        pl.semaphore_wait(barrier_sem, 2)

        def partial_top(c, j=None):
            start = c * m_per if j is None else c * m_per + j * rows
            xc = x_ref[pl.ds(start, half if j is None else rows), :]
            return jnp.dot(xc, w_ref[:, :], preferred_element_type=jnp.float32)

        def partial_bot(c, j=None):
            start = c * m_per + half if j is None else c * m_per + half + j * rows
            xc = x_ref[pl.ds(start, half if j is None else rows), :]
            return jnp.dot(xc, w_ref[:, :], preferred_element_type=jnp.float32)

        def sub_copy(comm, slot_s, slot_r, j, ssem, rsem, dev):
            return pltpu.make_async_remote_copy(
                src_ref=comm.at[slot_s, pl.ds(j * rows, rows), :],
                dst_ref=comm.at[slot_r, pl.ds(j * rows, rows), :],
                send_sem=ssem, recv_sem=rsem,
                device_id=(dev,), device_id_type=pl.DeviceIdType.MESH,
            )

        p_top = p_bot = None
        for s in range(N_DEV - 1):
            snd = (s + NSLOT - 1) % NSLOT
            rcv = s % NSLOT
            if s >= 2:
                pl.semaphore_wait(credit_r.at[rcv], 1)
                pl.semaphore_wait(credit_l.at[rcv], 1)
            sends = []
            for j in range(SUB):
                jr = pl.ds(j * rows, rows)
                if s == 0:
                    comm_r[snd, jr, :] = partial_top(left, j).astype(jnp.bfloat16)
                    comm_l[snd, jr, :] = partial_bot(right, j).astype(jnp.bfloat16)
                else:
                    sub_copy(comm_r, snd, snd, j, send_r.at[snd, j],
                             recv_r.at[snd, j], left).wait_recv()
                    comm_r[snd, jr, :] = (
                        comm_r[snd, jr, :].astype(jnp.float32)
                        + p_top[j * rows:(j + 1) * rows, :].astype(jnp.float32)
                    ).astype(jnp.bfloat16)
                    sub_copy(comm_l, snd, snd, j, send_l.at[snd, j],
                             recv_l.at[snd, j], right).wait_recv()
                    comm_l[snd, jr, :] = (
                        comm_l[snd, jr, :].astype(jnp.float32)
                        + p_bot[j * rows:(j + 1) * rows, :].astype(jnp.float32)
                    ).astype(jnp.bfloat16)
                r = sub_copy(comm_r, snd, rcv, j, send_r.at[snd, j],
                             recv_r.at[rcv, j], right)
                l = sub_copy(comm_l, snd, rcv, j, send_l.at[snd, j],
                             recv_l.at[rcv, j], left)
                r.start()
                l.start()
                sends.append((r, l))
            p_top = partial_top(
                lax.rem(my + 2 * N_DEV - s - 2, N_DEV)).astype(jnp.bfloat16)
            p_bot = partial_bot(
                lax.rem(my + s + 2, N_DEV)).astype(jnp.bfloat16)
            for r, l in sends:
                r.wait_send()
                l.wait_send()
            if s < N_DEV - 3:
                pl.semaphore_signal(
                    credit_r.at[snd], inc=1,
                    device_id=(left,), device_id_type=pl.DeviceIdType.MESH,
                )
                pl.semaphore_signal(
                    credit_l.at[snd], inc=1,
                    device_id=(right,), device_id_type=pl.DeviceIdType.MESH,
                )

        fin = (N_DEV - 2) % NSLOT
        for j in range(SUB):
            jr = pl.ds(j * rows, rows)
            sub_copy(comm_r, fin, fin, j, send_r.at[fin, j],
                     recv_r.at[fin, j], left).wait_recv()
            out_ref[jr, :] = (
                comm_r[fin, jr, :].astype(jnp.float32)
                + p_top[j * rows:(j + 1) * rows, :].astype(jnp.float32)
            )
            sub_copy(comm_l, fin, fin, j, send_l.at[fin, j],
                     recv_l.at[fin, j], right).wait_recv()
            out_ref[pl.ds(half + j * rows, rows), :] = (
                comm_l[fin, jr, :].astype(jnp.float32)
                + p_bot[j * rows:(j + 1) * rows, :].astype(jnp.float32)
            )

    return pl.pallas_call(
        body,
        out_shape=jax.ShapeDtypeStruct((m_per, n), jnp.float32),
        in_specs=[
            pl.BlockSpec(memory_space=pltpu.VMEM),
            pl.BlockSpec(memory_space=pltpu.VMEM),
        ],
        out_specs=pl.BlockSpec(memory_space=pltpu.VMEM),
        scratch_shapes=[
            pltpu.VMEM((NSLOT, half, n), jnp.bfloat16),
            pltpu.VMEM((NSLOT, half, n), jnp.bfloat16),
            pltpu.SemaphoreType.DMA((NSLOT, SUB)),
            pltpu.SemaphoreType.DMA((NSLOT, SUB)),
            pltpu.SemaphoreType.DMA((NSLOT, SUB)),
            pltpu.SemaphoreType.DMA((NSLOT, SUB)),
            pltpu.SemaphoreType.REGULAR((NSLOT,)),
            pltpu.SemaphoreType.REGULAR((NSLOT,)),
        ],
        compiler_params=pltpu.CompilerParams(
            collective_id=0,
            vmem_limit_bytes=100 * 1024 * 1024,
        ),
    )(x, w_mat)
